# baseline (speedup 1.0000x reference)
import functools

import jax
import jax.numpy as jnp
from jax import lax
from jax.experimental import pallas as pl
from jax.experimental.pallas import tpu as pltpu
from jax.experimental.pallas import tpu_sc as plsc

N = 10000
E = 320000
D_IN = 128
D_EDGE = 16
MSG = 32
HID = 16

NW = 32
LANES = 16
GCHUNK = 2000
SCHUNK = 6400
NEG = float("-inf")


def _g_body(xa_ref, xb_ref, wa_ref, ba_ref, wb_ref, bb_ref, ga_ref, gb_ref):
    ga_ref[...] = lax.dot_general(
        xa_ref[...], wa_ref[...], (((1,), (1,)), ((), ())),
        preferred_element_type=jnp.float32) + ba_ref[...]
    gb_ref[...] = lax.dot_general(
        xb_ref[...], wb_ref[...], (((1,), (1,)), ((), ())),
        preferred_element_type=jnp.float32) + bb_ref[...]


def _node_proj(x_A, x_B, W1x_ab, b1_ab, W1x_ba, b1_ba):
    return pl.pallas_call(
        _g_body,
        out_shape=[jax.ShapeDtypeStruct((N, HID), jnp.float32)] * 2,
    )(x_A, x_B, W1x_ab, b1_ab.reshape(1, HID), W1x_ba, b1_ba.reshape(1, HID))


def _msg_body(ef_ref, gat_ref, we_ref, w2_ref, b2_ref, mt_ref):
    h = lax.dot_general(ef_ref[...], we_ref[...], (((1,), (1,)), ((), ())),
                        preferred_element_type=jnp.float32)
    h = jnp.maximum(h + gat_ref[...], 0.0)
    mt_ref[...] = lax.dot_general(
        w2_ref[...], h, (((1,), (1,)), ((), ())),
        preferred_element_type=jnp.float32) + b2_ref[...]


def _msg_mlp(efeat, gat, W1e, W2, b2):
    be = 16000
    grid = E // be
    return pl.pallas_call(
        _msg_body,
        grid=(grid,),
        in_specs=[
            pl.BlockSpec((be, D_EDGE), lambda j: (j, 0)),
            pl.BlockSpec((be, HID), lambda j: (j, 0)),
            pl.BlockSpec((HID, D_EDGE), lambda j: (0, 0)),
            pl.BlockSpec((MSG, HID), lambda j: (0, 0)),
            pl.BlockSpec((MSG, 1), lambda j: (0, 0)),
        ],
        out_specs=pl.BlockSpec((MSG, be), lambda j: (0, j)),
        out_shape=jax.ShapeDtypeStruct((MSG, E), jnp.float32),
    )(efeat, gat, W1e, W2, b2.reshape(MSG, 1))


def _udt_body(x_ref, rt_ref, w1x_ref, w1r_ref, b1_ref, w2_ref, b2_ref, o_ref):
    rt = rt_ref[...]
    rt = jnp.where(jnp.isneginf(rt), 0.0, rt)
    h = lax.dot_general(x_ref[...], w1x_ref[...], (((1,), (1,)), ((), ())),
                        preferred_element_type=jnp.float32)
    h = h + lax.dot_general(rt, w1r_ref[...], (((0,), (1,)), ((), ())),
                            preferred_element_type=jnp.float32)
    h = jnp.maximum(h + b1_ref[...], 0.0)
    o_ref[...] = lax.dot_general(
        h, w2_ref[...], (((1,), (1,)), ((), ())),
        preferred_element_type=jnp.float32) + b2_ref[...]


def _udt_mlp(x_dst, r_t, uW1x, uW1r, ub1, uW2, ub2):
    return pl.pallas_call(
        _udt_body,
        out_shape=jax.ShapeDtypeStruct((N, D_IN), jnp.float32),
    )(x_dst, r_t, uW1x, uW1r, ub1.reshape(1, HID), uW2, ub2.reshape(1, D_IN))


_MESH = None


def _mesh():
    global _MESH
    if _MESH is None:
        _MESH = plsc.VectorSubcoreMesh(core_axis_name="c", subcore_axis_name="s")
    return _MESH


def _gather_kernel(ga_hbm, sa_hbm, gb_hbm, sb_hbm, oa_hbm, ob_hbm,
                   idx_v, rows_v, sem):
    wid = lax.axis_index("s") * 2 + lax.axis_index("c")
    per_w = E // NW
    nchunk = per_w // GCHUNK

    for g_hbm, s_hbm, o_hbm in ((ga_hbm, sa_hbm, oa_hbm),
                                (gb_hbm, sb_hbm, ob_hbm)):
        def body(k, _, g_hbm=g_hbm, s_hbm=s_hbm, o_hbm=o_hbm):
            base = wid * per_w + k * GCHUNK
            pltpu.sync_copy(s_hbm.at[pl.ds(base, GCHUNK)], idx_v)
            pltpu.async_copy(g_hbm.at[idx_v], rows_v, sem).wait()
            pltpu.sync_copy(rows_v, o_hbm.at[pl.ds(base, GCHUNK)])
            return _
        lax.fori_loop(0, nchunk, body, None)


def _sc_gather(g_ab, src_ab, g_ba, src_ba):
    k = pl.kernel(
        _gather_kernel,
        out_type=[jax.ShapeDtypeStruct((E, HID), jnp.float32)] * 2,
        mesh=_mesh(),
        compiler_params=pltpu.CompilerParams(use_tc_tiling_on_sc=False),
        scratch_types=[
            pltpu.VMEM((GCHUNK,), jnp.int32),
            pltpu.VMEM((GCHUNK, HID), jnp.float32),
            pltpu.SemaphoreType.DMA,
        ],
    )
    return k(g_ab, src_ab, g_ba, src_ba)


OCC0 = 0
NACC = 8
VPC = SCHUNK // LANES
GPC = VPC // NACC


def _scatter_kernel(ma_hbm, da_hbm, mb_hbm, db_hbm, ra_hbm, rb_hbm,
                    dst0, dst1, val0, val1,
                    a0, a1, a2, a3, a4, a5, a6, a7,
                    sd0, sd1, sv0, sv1):
    wid = lax.axis_index("s") * 2 + lax.axis_index("c")
    nchunk = E // SCHUNK
    accs = (a0, a1, a2, a3, a4, a5, a6, a7)
    dbuf = (dst0, dst1)
    vbuf = (val0, val1)
    dsem = (sd0, sd1)
    vsem = (sv0, sv1)
    neg16 = jnp.full((LANES,), NEG, jnp.float32)

    def dst_slice(d_hbm, k):
        return d_hbm.at[pl.ds(k * SCHUNK, SCHUNK)]

    def val_slice(m_hbm, k):
        return m_hbm.at[pl.ds(wid * E + k * SCHUNK, SCHUNK)]

    for m_hbm, d_hbm, r_hbm in ((ma_hbm, da_hbm, ra_hbm),
                                (mb_hbm, db_hbm, rb_hbm)):
        def init(i, _):
            for a in accs:
                a[pl.ds(i * LANES, LANES)] = neg16
            return _
        lax.fori_loop(0, N // LANES, init, None)

        for b in (0, 1):
            pltpu.async_copy(dst_slice(d_hbm, b), dbuf[b], dsem[b])
            pltpu.async_copy(val_slice(m_hbm, b), vbuf[b], vsem[b])

        def do_chunk(b, k, m_hbm=m_hbm, d_hbm=d_hbm):
            pltpu.make_async_copy(dst_slice(d_hbm, k), dbuf[b],
                                  dsem[b]).wait()
            pltpu.make_async_copy(val_slice(m_hbm, k), vbuf[b],
                                  vsem[b]).wait()

            def group(g, dirty):
                for sub in range(NACC):
                    i = g * NACC + sub
                    idx = dbuf[b][pl.ds(i * LANES, LANES)]
                    val = vbuf[b][pl.ds(i * LANES, LANES)]
                    occ, _ = plsc.scan_count(idx)
                    a = accs[sub]
                    m0 = occ == OCC0
                    old0 = plsc.load_gather(a, [idx], mask=m0)
                    plsc.store_scatter(a, [idx], jnp.maximum(old0, val),
                                       mask=m0)
                    m1 = occ == OCC0 + 1
                    old1 = plsc.load_gather(a, [idx], mask=m1)
                    plsc.store_scatter(a, [idx], jnp.maximum(old1, val),
                                       mask=m1)
                    dirty = jnp.maximum(dirty, occ)
                return dirty
            dirty = lax.fori_loop(0, GPC, group,
                                  jnp.zeros((LANES,), jnp.int32))

            def slow():
                def redo(i, carry):
                    idx = dbuf[b][pl.ds(i * LANES, LANES)]
                    val = vbuf[b][pl.ds(i * LANES, LANES)]
                    occ, _last = plsc.scan_count(idx)
                    for lvl in range(LANES):
                        m = occ == OCC0 + lvl
                        old = plsc.load_gather(a0, [idx], mask=m)
                        plsc.store_scatter(a0, [idx],
                                           jnp.maximum(old, val), mask=m)
                    return carry
                lax.fori_loop(0, VPC, redo, None)

            pl.when(jnp.max(dirty) >= OCC0 + 2)(slow)

            @pl.when(k + 2 < nchunk)
            def _prefetch():
                pltpu.async_copy(dst_slice(d_hbm, k + 2), dbuf[b], dsem[b])
                pltpu.async_copy(val_slice(m_hbm, k + 2), vbuf[b], vsem[b])

        def pair(p, _):
            do_chunk(0, p * 2)
            do_chunk(1, p * 2 + 1)
            return _
        lax.fori_loop(0, nchunk // 2, pair, None)

        def merge(i, _):
            sl = pl.ds(i * LANES, LANES)
            v = a0[sl]
            for a in accs[1:]:
                v = jnp.maximum(v, a[sl])
            a0[sl] = v
            return _
        lax.fori_loop(0, N // LANES, merge, None)
        pltpu.sync_copy(a0, r_hbm.at[pl.ds(wid * N, N)])


def _sc_segment_max(m_t_ab, dst_ab, m_t_ba, dst_ba):
    k = pl.kernel(
        _scatter_kernel,
        out_type=[jax.ShapeDtypeStruct((MSG * N,), jnp.float32)] * 2,
        mesh=_mesh(),
        compiler_params=pltpu.CompilerParams(use_tc_tiling_on_sc=False,
                                             needs_layout_passes=False),
        scratch_types=(
            [pltpu.VMEM((SCHUNK,), jnp.int32)] * 2
            + [pltpu.VMEM((SCHUNK,), jnp.float32)] * 2
            + [pltpu.VMEM((N,), jnp.float32)] * NACC
            + [pltpu.SemaphoreType.DMA] * 4
        ),
    )
    return k(m_t_ab, dst_ab, m_t_ba, dst_ba)


def kernel(x_A, x_B, edge_index_ab, edge_feat_ab, edge_index_ba, edge_feat_ba,
           ab_msg_W1, ab_msg_b1, ab_msg_W2, ab_msg_b2,
           ab_udt_W1, ab_udt_b1, ab_udt_W2, ab_udt_b2,
           ba_msg_W1, ba_msg_b1, ba_msg_W2, ba_msg_b2,
           ba_udt_W1, ba_udt_b1, ba_udt_W2, ba_udt_b2):
    src_ab = edge_index_ab[0].astype(jnp.int32)
    dst_ab = edge_index_ab[1].astype(jnp.int32)
    src_ba = edge_index_ba[0].astype(jnp.int32)
    dst_ba = edge_index_ba[1].astype(jnp.int32)

    ab_W1e, ab_W1x = ab_msg_W1[:, :D_EDGE], ab_msg_W1[:, D_EDGE:]
    ba_W1e, ba_W1x = ba_msg_W1[:, :D_EDGE], ba_msg_W1[:, D_EDGE:]
    ab_uW1x, ab_uW1r = ab_udt_W1[:, :D_IN], ab_udt_W1[:, D_IN:]
    ba_uW1x, ba_uW1r = ba_udt_W1[:, :D_IN], ba_udt_W1[:, D_IN:]

    g_ab, g_ba = _node_proj(x_A, x_B, ab_W1x, ab_msg_b1, ba_W1x, ba_msg_b1)
    gat_ab, gat_ba = _sc_gather(g_ab, src_ab, g_ba, src_ba)

    mt_ab = _msg_mlp(edge_feat_ab, gat_ab, ab_W1e, ab_msg_W2, ab_msg_b2)
    mt_ba = _msg_mlp(edge_feat_ba, gat_ba, ba_W1e, ba_msg_W2, ba_msg_b2)

    r_ab, r_ba = _sc_segment_max(mt_ab.reshape(-1), dst_ab,
                                 mt_ba.reshape(-1), dst_ba)
    r_ab = r_ab.reshape(MSG, N)
    r_ba = r_ba.reshape(MSG, N)

    out_B = _udt_mlp(x_B, r_ab, ab_uW1x, ab_uW1r, ab_udt_b1,
                     ab_udt_W2, ab_udt_b2)
    out_A = _udt_mlp(x_A, r_ba, ba_uW1x, ba_uW1r, ba_udt_b1,
                     ba_udt_W2, ba_udt_b2)
    return (out_A, out_B)


# speedup vs baseline: 3.0501x; 3.0501x over previous
import functools

import jax
import jax.numpy as jnp
from jax import lax
from jax.experimental import pallas as pl
from jax.experimental.pallas import tpu as pltpu
from jax.experimental.pallas import tpu_sc as plsc

N = 10000
E = 320000
D_IN = 128
D_EDGE = 16
MSG = 32
HID = 16

NW = 32
LANES = 16
GCHUNK = 2000
SCHUNK = 6400
NEG = float("-inf")


def _g_body(xa_ref, xb_ref, wa_ref, ba_ref, wb_ref, bb_ref, ga_ref, gb_ref):
    ga_ref[...] = lax.dot_general(
        xa_ref[...], wa_ref[...], (((1,), (1,)), ((), ())),
        preferred_element_type=jnp.float32) + ba_ref[...]
    gb_ref[...] = lax.dot_general(
        xb_ref[...], wb_ref[...], (((1,), (1,)), ((), ())),
        preferred_element_type=jnp.float32) + bb_ref[...]


def _node_proj(x_A, x_B, W1x_ab, b1_ab, W1x_ba, b1_ba):
    return pl.pallas_call(
        _g_body,
        out_shape=[jax.ShapeDtypeStruct((N, HID), jnp.float32)] * 2,
    )(x_A, x_B, W1x_ab, b1_ab.reshape(1, HID), W1x_ba, b1_ba.reshape(1, HID))


def _msg_body(ef_ref, gat_ref, we_ref, w2_ref, b2_ref, mt_ref):
    h = lax.dot_general(ef_ref[...], we_ref[...], (((1,), (1,)), ((), ())),
                        preferred_element_type=jnp.float32)
    h = jnp.maximum(h + gat_ref[...], 0.0)
    mt_ref[...] = lax.dot_general(
        w2_ref[...], h, (((1,), (1,)), ((), ())),
        preferred_element_type=jnp.float32) + b2_ref[...]


def _msg_mlp(efeat, gat, W1e, W2, b2):
    be = 16000
    grid = E // be
    return pl.pallas_call(
        _msg_body,
        grid=(grid,),
        in_specs=[
            pl.BlockSpec((be, D_EDGE), lambda j: (j, 0)),
            pl.BlockSpec((be, HID), lambda j: (j, 0)),
            pl.BlockSpec((HID, D_EDGE), lambda j: (0, 0)),
            pl.BlockSpec((MSG, HID), lambda j: (0, 0)),
            pl.BlockSpec((MSG, 1), lambda j: (0, 0)),
        ],
        out_specs=pl.BlockSpec((MSG, be), lambda j: (0, j)),
        out_shape=jax.ShapeDtypeStruct((MSG, E), jnp.float32),
    )(efeat, gat, W1e, W2, b2.reshape(MSG, 1))


def _udt_body(x_ref, rt_ref, w1x_ref, w1r_ref, b1_ref, w2_ref, b2_ref, o_ref):
    rt = rt_ref[...]
    rt = jnp.where(jnp.isneginf(rt), 0.0, rt)
    h = lax.dot_general(x_ref[...], w1x_ref[...], (((1,), (1,)), ((), ())),
                        preferred_element_type=jnp.float32)
    h = h + lax.dot_general(rt, w1r_ref[...], (((0,), (1,)), ((), ())),
                            preferred_element_type=jnp.float32)
    h = jnp.maximum(h + b1_ref[...], 0.0)
    o_ref[...] = lax.dot_general(
        h, w2_ref[...], (((1,), (1,)), ((), ())),
        preferred_element_type=jnp.float32) + b2_ref[...]


def _udt_mlp(x_dst, r_t, uW1x, uW1r, ub1, uW2, ub2):
    return pl.pallas_call(
        _udt_body,
        out_shape=jax.ShapeDtypeStruct((N, D_IN), jnp.float32),
    )(x_dst, r_t, uW1x, uW1r, ub1.reshape(1, HID), uW2, ub2.reshape(1, D_IN))


_MESH = None


def _mesh():
    global _MESH
    if _MESH is None:
        _MESH = plsc.VectorSubcoreMesh(core_axis_name="c", subcore_axis_name="s")
    return _MESH


def _gather_kernel(ga_hbm, sa_hbm, gb_hbm, sb_hbm, oa_hbm, ob_hbm,
                   idx_v, rows_v, sem):
    wid = lax.axis_index("s") * 2 + lax.axis_index("c")
    per_w = E // NW
    nchunk = per_w // GCHUNK

    for g_hbm, s_hbm, o_hbm in ((ga_hbm, sa_hbm, oa_hbm),
                                (gb_hbm, sb_hbm, ob_hbm)):
        def body(k, _, g_hbm=g_hbm, s_hbm=s_hbm, o_hbm=o_hbm):
            base = wid * per_w + k * GCHUNK
            pltpu.sync_copy(s_hbm.at[pl.ds(base, GCHUNK)], idx_v)
            pltpu.async_copy(g_hbm.at[idx_v], rows_v, sem).wait()
            pltpu.sync_copy(rows_v, o_hbm.at[pl.ds(base, GCHUNK)])
            return _
        lax.fori_loop(0, nchunk, body, None)


def _sc_gather(g_ab, src_ab, g_ba, src_ba):
    k = pl.kernel(
        _gather_kernel,
        out_type=[jax.ShapeDtypeStruct((E, HID), jnp.float32)] * 2,
        mesh=_mesh(),
        compiler_params=pltpu.CompilerParams(use_tc_tiling_on_sc=False),
        scratch_types=[
            pltpu.VMEM((GCHUNK,), jnp.int32),
            pltpu.VMEM((GCHUNK, HID), jnp.float32),
            pltpu.SemaphoreType.DMA,
        ],
    )
    return k(g_ab, src_ab, g_ba, src_ba)


OCC0 = 1
NACC = 8
VPC = SCHUNK // LANES
GPC = VPC // NACC


def _scatter_kernel(ma_hbm, da_hbm, mb_hbm, db_hbm, ra_hbm, rb_hbm,
                    dst0, dst1, val0, val1,
                    a0, a1, a2, a3, a4, a5, a6, a7,
                    sd0, sd1, sv0, sv1):
    wid = lax.axis_index("s") * 2 + lax.axis_index("c")
    nchunk = E // SCHUNK
    accs = (a0, a1, a2, a3, a4, a5, a6, a7)
    dbuf = (dst0, dst1)
    vbuf = (val0, val1)
    dsem = (sd0, sd1)
    vsem = (sv0, sv1)
    neg16 = jnp.full((LANES,), NEG, jnp.float32)

    def dst_slice(d_hbm, k):
        return d_hbm.at[pl.ds(k * SCHUNK, SCHUNK)]

    def val_slice(m_hbm, k):
        return m_hbm.at[pl.ds(wid * E + k * SCHUNK, SCHUNK)]

    for m_hbm, d_hbm, r_hbm in ((ma_hbm, da_hbm, ra_hbm),
                                (mb_hbm, db_hbm, rb_hbm)):
        def init(i, _):
            for a in accs:
                a[pl.ds(i * LANES, LANES)] = neg16
            return _
        lax.fori_loop(0, N // LANES, init, None)

        for b in (0, 1):
            pltpu.async_copy(dst_slice(d_hbm, b), dbuf[b], dsem[b])
            pltpu.async_copy(val_slice(m_hbm, b), vbuf[b], vsem[b])

        def do_chunk(b, k, m_hbm=m_hbm, d_hbm=d_hbm):
            pltpu.make_async_copy(dst_slice(d_hbm, k), dbuf[b],
                                  dsem[b]).wait()
            pltpu.make_async_copy(val_slice(m_hbm, k), vbuf[b],
                                  vsem[b]).wait()

            def group(g, dirty):
                for sub in range(NACC):
                    i = g * NACC + sub
                    idx = dbuf[b][pl.ds(i * LANES, LANES)]
                    val = vbuf[b][pl.ds(i * LANES, LANES)]
                    occ, _ = plsc.scan_count(idx)
                    a = accs[sub]
                    m0 = occ == OCC0
                    old0 = plsc.load_gather(a, [idx], mask=m0)
                    plsc.store_scatter(a, [idx], jnp.maximum(old0, val),
                                       mask=m0)
                    m1 = occ == OCC0 + 1
                    old1 = plsc.load_gather(a, [idx], mask=m1)
                    plsc.store_scatter(a, [idx], jnp.maximum(old1, val),
                                       mask=m1)
                    dirty = jnp.maximum(dirty, occ)
                return dirty
            dirty = lax.fori_loop(0, GPC, group,
                                  jnp.zeros((LANES,), jnp.int32))

            def slow():
                def redo(i, carry):
                    idx = dbuf[b][pl.ds(i * LANES, LANES)]
                    val = vbuf[b][pl.ds(i * LANES, LANES)]
                    occ, _last = plsc.scan_count(idx)
                    for lvl in range(LANES):
                        m = occ == OCC0 + lvl
                        old = plsc.load_gather(a0, [idx], mask=m)
                        plsc.store_scatter(a0, [idx],
                                           jnp.maximum(old, val), mask=m)
                    return carry
                lax.fori_loop(0, VPC, redo, None)

            pl.when(jnp.max(dirty) >= OCC0 + 2)(slow)

            @pl.when(k + 2 < nchunk)
            def _prefetch():
                pltpu.async_copy(dst_slice(d_hbm, k + 2), dbuf[b], dsem[b])
                pltpu.async_copy(val_slice(m_hbm, k + 2), vbuf[b], vsem[b])

        def pair(p, _):
            do_chunk(0, p * 2)
            do_chunk(1, p * 2 + 1)
            return _
        lax.fori_loop(0, nchunk // 2, pair, None)

        def merge(i, _):
            sl = pl.ds(i * LANES, LANES)
            v = a0[sl]
            for a in accs[1:]:
                v = jnp.maximum(v, a[sl])
            a0[sl] = v
            return _
        lax.fori_loop(0, N // LANES, merge, None)
        pltpu.sync_copy(a0, r_hbm.at[pl.ds(wid * N, N)])


def _sc_segment_max(m_t_ab, dst_ab, m_t_ba, dst_ba):
    k = pl.kernel(
        _scatter_kernel,
        out_type=[jax.ShapeDtypeStruct((MSG * N,), jnp.float32)] * 2,
        mesh=_mesh(),
        compiler_params=pltpu.CompilerParams(use_tc_tiling_on_sc=False,
                                             needs_layout_passes=False),
        scratch_types=(
            [pltpu.VMEM((SCHUNK,), jnp.int32)] * 2
            + [pltpu.VMEM((SCHUNK,), jnp.float32)] * 2
            + [pltpu.VMEM((N,), jnp.float32)] * NACC
            + [pltpu.SemaphoreType.DMA] * 4
        ),
    )
    return k(m_t_ab, dst_ab, m_t_ba, dst_ba)


def kernel(x_A, x_B, edge_index_ab, edge_feat_ab, edge_index_ba, edge_feat_ba,
           ab_msg_W1, ab_msg_b1, ab_msg_W2, ab_msg_b2,
           ab_udt_W1, ab_udt_b1, ab_udt_W2, ab_udt_b2,
           ba_msg_W1, ba_msg_b1, ba_msg_W2, ba_msg_b2,
           ba_udt_W1, ba_udt_b1, ba_udt_W2, ba_udt_b2):
    src_ab = edge_index_ab[0].astype(jnp.int32)
    dst_ab = edge_index_ab[1].astype(jnp.int32)
    src_ba = edge_index_ba[0].astype(jnp.int32)
    dst_ba = edge_index_ba[1].astype(jnp.int32)

    ab_W1e, ab_W1x = ab_msg_W1[:, :D_EDGE], ab_msg_W1[:, D_EDGE:]
    ba_W1e, ba_W1x = ba_msg_W1[:, :D_EDGE], ba_msg_W1[:, D_EDGE:]
    ab_uW1x, ab_uW1r = ab_udt_W1[:, :D_IN], ab_udt_W1[:, D_IN:]
    ba_uW1x, ba_uW1r = ba_udt_W1[:, :D_IN], ba_udt_W1[:, D_IN:]

    g_ab, g_ba = _node_proj(x_A, x_B, ab_W1x, ab_msg_b1, ba_W1x, ba_msg_b1)
    gat_ab, gat_ba = _sc_gather(g_ab, src_ab, g_ba, src_ba)

    mt_ab = _msg_mlp(edge_feat_ab, gat_ab, ab_W1e, ab_msg_W2, ab_msg_b2)
    mt_ba = _msg_mlp(edge_feat_ba, gat_ba, ba_W1e, ba_msg_W2, ba_msg_b2)

    r_ab, r_ba = _sc_segment_max(mt_ab.reshape(-1), dst_ab,
                                 mt_ba.reshape(-1), dst_ba)
    r_ab = r_ab.reshape(MSG, N)
    r_ba = r_ba.reshape(MSG, N)

    out_B = _udt_mlp(x_B, r_ab, ab_uW1x, ab_uW1r, ab_udt_b1,
                     ab_udt_W2, ab_udt_b2)
    out_A = _udt_mlp(x_A, r_ba, ba_uW1x, ba_uW1r, ba_udt_b1,
                     ba_udt_W2, ba_udt_b2)
    return (out_A, out_B)


# speedup vs baseline: 3.2371x; 1.0613x over previous
import functools

import jax
import jax.numpy as jnp
from jax import lax
from jax.experimental import pallas as pl
from jax.experimental.pallas import tpu as pltpu
from jax.experimental.pallas import tpu_sc as plsc

N = 10000
E = 320000
D_IN = 128
D_EDGE = 16
MSG = 32
HID = 16

NW = 32
LANES = 16
GCHUNK = 2000
SCHUNK = 6400
NEG = float("-inf")


def _g_body(xa_ref, xb_ref, wa_ref, ba_ref, wb_ref, bb_ref, ga_ref, gb_ref):
    ga_ref[...] = lax.dot_general(
        xa_ref[...], wa_ref[...], (((1,), (1,)), ((), ())),
        preferred_element_type=jnp.float32) + ba_ref[...]
    gb_ref[...] = lax.dot_general(
        xb_ref[...], wb_ref[...], (((1,), (1,)), ((), ())),
        preferred_element_type=jnp.float32) + bb_ref[...]


def _node_proj(x_A, x_B, W1x_ab, b1_ab, W1x_ba, b1_ba):
    return pl.pallas_call(
        _g_body,
        out_shape=[jax.ShapeDtypeStruct((N, HID), jnp.float32)] * 2,
    )(x_A, x_B, W1x_ab, b1_ab.reshape(1, HID), W1x_ba, b1_ba.reshape(1, HID))


def _msg_body(ef_ref, gat_ref, we_ref, w2_ref, b2_ref, mt_ref):
    h = lax.dot_general(ef_ref[...], we_ref[...], (((1,), (1,)), ((), ())),
                        preferred_element_type=jnp.float32)
    h = jnp.maximum(h + gat_ref[...], 0.0)
    mt_ref[...] = lax.dot_general(
        w2_ref[...], h, (((1,), (1,)), ((), ())),
        preferred_element_type=jnp.float32) + b2_ref[...]


def _msg_mlp(efeat, gat, W1e, W2, b2):
    be = 16000
    grid = E // be
    return pl.pallas_call(
        _msg_body,
        grid=(grid,),
        in_specs=[
            pl.BlockSpec((be, D_EDGE), lambda j: (j, 0)),
            pl.BlockSpec((be, HID), lambda j: (j, 0)),
            pl.BlockSpec((HID, D_EDGE), lambda j: (0, 0)),
            pl.BlockSpec((MSG, HID), lambda j: (0, 0)),
            pl.BlockSpec((MSG, 1), lambda j: (0, 0)),
        ],
        out_specs=pl.BlockSpec((MSG, be), lambda j: (0, j)),
        out_shape=jax.ShapeDtypeStruct((MSG, E), jnp.float32),
    )(efeat, gat, W1e, W2, b2.reshape(MSG, 1))


def _udt_body(x_ref, rt_ref, w1x_ref, w1r_ref, b1_ref, w2_ref, b2_ref, o_ref):
    rt = rt_ref[...]
    rt = jnp.where(jnp.isneginf(rt), 0.0, rt)
    h = lax.dot_general(x_ref[...], w1x_ref[...], (((1,), (1,)), ((), ())),
                        preferred_element_type=jnp.float32)
    h = h + lax.dot_general(rt, w1r_ref[...], (((0,), (1,)), ((), ())),
                            preferred_element_type=jnp.float32)
    h = jnp.maximum(h + b1_ref[...], 0.0)
    o_ref[...] = lax.dot_general(
        h, w2_ref[...], (((1,), (1,)), ((), ())),
        preferred_element_type=jnp.float32) + b2_ref[...]


def _udt_mlp(x_dst, r_t, uW1x, uW1r, ub1, uW2, ub2):
    return pl.pallas_call(
        _udt_body,
        out_shape=jax.ShapeDtypeStruct((N, D_IN), jnp.float32),
    )(x_dst, r_t, uW1x, uW1r, ub1.reshape(1, HID), uW2, ub2.reshape(1, D_IN))


_MESH = None


def _mesh():
    global _MESH
    if _MESH is None:
        _MESH = plsc.VectorSubcoreMesh(core_axis_name="c", subcore_axis_name="s")
    return _MESH


def _gather_kernel(ga_hbm, sa_hbm, gb_hbm, sb_hbm, oa_hbm, ob_hbm,
                   idx_v, rows_v, sem):
    wid = lax.axis_index("s") * 2 + lax.axis_index("c")
    per_w = E // NW
    nchunk = per_w // GCHUNK

    for g_hbm, s_hbm, o_hbm in ((ga_hbm, sa_hbm, oa_hbm),
                                (gb_hbm, sb_hbm, ob_hbm)):
        def body(k, _, g_hbm=g_hbm, s_hbm=s_hbm, o_hbm=o_hbm):
            base = wid * per_w + k * GCHUNK
            pltpu.sync_copy(s_hbm.at[pl.ds(base, GCHUNK)], idx_v)
            pltpu.async_copy(g_hbm.at[idx_v], rows_v, sem).wait()
            pltpu.sync_copy(rows_v, o_hbm.at[pl.ds(base, GCHUNK)])
            return _
        lax.fori_loop(0, nchunk, body, None)


def _sc_gather(g_ab, src_ab, g_ba, src_ba):
    k = pl.kernel(
        _gather_kernel,
        out_type=[jax.ShapeDtypeStruct((E, HID), jnp.float32)] * 2,
        mesh=_mesh(),
        compiler_params=pltpu.CompilerParams(use_tc_tiling_on_sc=False),
        scratch_types=[
            pltpu.VMEM((GCHUNK,), jnp.int32),
            pltpu.VMEM((GCHUNK, HID), jnp.float32),
            pltpu.SemaphoreType.DMA,
        ],
    )
    return k(g_ab, src_ab, g_ba, src_ba)


OCC0 = 1
NACC = 8
VPC = SCHUNK // LANES
GPC = VPC // NACC


def _scatter_kernel(ma_hbm, da_hbm, mb_hbm, db_hbm, ra_hbm, rb_hbm,
                    dst0, dst1, val0, val1,
                    a0, a1, a2, a3, a4, a5, a6, a7,
                    sd0, sd1, sv0, sv1):
    wid = lax.axis_index("s") * 2 + lax.axis_index("c")
    nchunk = E // SCHUNK
    accs = (a0, a1, a2, a3, a4, a5, a6, a7)
    dbuf = (dst0, dst1)
    vbuf = (val0, val1)
    dsem = (sd0, sd1)
    vsem = (sv0, sv1)
    neg16 = jnp.full((LANES,), NEG, jnp.float32)
    one16 = jnp.ones((LANES,), jnp.int32)
    zero16 = jnp.zeros((LANES,), jnp.int32)

    def dst_slice(d_hbm, k):
        return d_hbm.at[pl.ds(k * SCHUNK, SCHUNK)]

    def val_slice(m_hbm, k):
        return m_hbm.at[pl.ds(wid * E + k * SCHUNK, SCHUNK)]

    for m_hbm, d_hbm, r_hbm in ((ma_hbm, da_hbm, ra_hbm),
                                (mb_hbm, db_hbm, rb_hbm)):
        def init(i, _):
            for a in accs:
                a[pl.ds(i * LANES, LANES)] = neg16
            return _
        lax.fori_loop(0, N // LANES, init, None)

        for b in (0, 1):
            pltpu.async_copy(dst_slice(d_hbm, b), dbuf[b], dsem[b])
            pltpu.async_copy(val_slice(m_hbm, b), vbuf[b], vsem[b])

        def do_chunk(b, k, m_hbm=m_hbm, d_hbm=d_hbm):
            pltpu.make_async_copy(dst_slice(d_hbm, k), dbuf[b],
                                  dsem[b]).wait()
            pltpu.make_async_copy(val_slice(m_hbm, k), vbuf[b],
                                  vsem[b]).wait()

            def group(g, dirty):
                for sub in range(NACC):
                    i = g * NACC + sub
                    idx = dbuf[b][pl.ds(i * LANES, LANES)]
                    val = vbuf[b][pl.ds(i * LANES, LANES)]
                    a = accs[sub]
                    old0 = plsc.load_gather(a, [idx])
                    new0 = jnp.maximum(old0, val)
                    plsc.store_scatter(a, [idx], new0)
                    chk0 = plsc.load_gather(a, [idx])
                    m1 = chk0 < new0
                    old1 = plsc.load_gather(a, [idx], mask=m1)
                    new1 = jnp.maximum(old1, val)
                    plsc.store_scatter(a, [idx], new1, mask=m1)
                    chk1 = plsc.load_gather(a, [idx], mask=m1)
                    lost = m1 & (chk1 < new1)
                    dirty = jnp.maximum(dirty, jnp.where(lost, one16, zero16))
                return dirty
            dirty = lax.fori_loop(0, GPC, group,
                                  jnp.zeros((LANES,), jnp.int32))

            def slow():
                def redo(i, carry):
                    idx = dbuf[b][pl.ds(i * LANES, LANES)]
                    val = vbuf[b][pl.ds(i * LANES, LANES)]
                    occ, _last = plsc.scan_count(idx)
                    for lvl in range(LANES):
                        m = occ == OCC0 + lvl
                        old = plsc.load_gather(a0, [idx], mask=m)
                        plsc.store_scatter(a0, [idx],
                                           jnp.maximum(old, val), mask=m)
                    return carry
                lax.fori_loop(0, VPC, redo, None)

            pl.when(jnp.max(dirty) > 0)(slow)

            @pl.when(k + 2 < nchunk)
            def _prefetch():
                pltpu.async_copy(dst_slice(d_hbm, k + 2), dbuf[b], dsem[b])
                pltpu.async_copy(val_slice(m_hbm, k + 2), vbuf[b], vsem[b])

        def pair(p, _):
            do_chunk(0, p * 2)
            do_chunk(1, p * 2 + 1)
            return _
        lax.fori_loop(0, nchunk // 2, pair, None)

        def merge(i, _):
            sl = pl.ds(i * LANES, LANES)
            v = a0[sl]
            for a in accs[1:]:
                v = jnp.maximum(v, a[sl])
            a0[sl] = v
            return _
        lax.fori_loop(0, N // LANES, merge, None)
        pltpu.sync_copy(a0, r_hbm.at[pl.ds(wid * N, N)])


def _sc_segment_max(m_t_ab, dst_ab, m_t_ba, dst_ba):
    k = pl.kernel(
        _scatter_kernel,
        out_type=[jax.ShapeDtypeStruct((MSG * N,), jnp.float32)] * 2,
        mesh=_mesh(),
        compiler_params=pltpu.CompilerParams(use_tc_tiling_on_sc=False,
                                             needs_layout_passes=False),
        scratch_types=(
            [pltpu.VMEM((SCHUNK,), jnp.int32)] * 2
            + [pltpu.VMEM((SCHUNK,), jnp.float32)] * 2
            + [pltpu.VMEM((N,), jnp.float32)] * NACC
            + [pltpu.SemaphoreType.DMA] * 4
        ),
    )
    return k(m_t_ab, dst_ab, m_t_ba, dst_ba)


def kernel(x_A, x_B, edge_index_ab, edge_feat_ab, edge_index_ba, edge_feat_ba,
           ab_msg_W1, ab_msg_b1, ab_msg_W2, ab_msg_b2,
           ab_udt_W1, ab_udt_b1, ab_udt_W2, ab_udt_b2,
           ba_msg_W1, ba_msg_b1, ba_msg_W2, ba_msg_b2,
           ba_udt_W1, ba_udt_b1, ba_udt_W2, ba_udt_b2):
    src_ab = edge_index_ab[0].astype(jnp.int32)
    dst_ab = edge_index_ab[1].astype(jnp.int32)
    src_ba = edge_index_ba[0].astype(jnp.int32)
    dst_ba = edge_index_ba[1].astype(jnp.int32)

    ab_W1e, ab_W1x = ab_msg_W1[:, :D_EDGE], ab_msg_W1[:, D_EDGE:]
    ba_W1e, ba_W1x = ba_msg_W1[:, :D_EDGE], ba_msg_W1[:, D_EDGE:]
    ab_uW1x, ab_uW1r = ab_udt_W1[:, :D_IN], ab_udt_W1[:, D_IN:]
    ba_uW1x, ba_uW1r = ba_udt_W1[:, :D_IN], ba_udt_W1[:, D_IN:]

    g_ab, g_ba = _node_proj(x_A, x_B, ab_W1x, ab_msg_b1, ba_W1x, ba_msg_b1)
    gat_ab, gat_ba = _sc_gather(g_ab, src_ab, g_ba, src_ba)

    mt_ab = _msg_mlp(edge_feat_ab, gat_ab, ab_W1e, ab_msg_W2, ab_msg_b2)
    mt_ba = _msg_mlp(edge_feat_ba, gat_ba, ba_W1e, ba_msg_W2, ba_msg_b2)

    r_ab, r_ba = _sc_segment_max(mt_ab.reshape(-1), dst_ab,
                                 mt_ba.reshape(-1), dst_ba)
    r_ab = r_ab.reshape(MSG, N)
    r_ba = r_ba.reshape(MSG, N)

    out_B = _udt_mlp(x_B, r_ab, ab_uW1x, ab_uW1r, ab_udt_b1,
                     ab_udt_W2, ab_udt_b2)
    out_A = _udt_mlp(x_A, r_ba, ba_uW1x, ba_uW1r, ba_udt_b1,
                     ba_udt_W2, ba_udt_b2)
    return (out_A, out_B)
